# baseline (speedup 1.0000x reference)
import jax
import jax.numpy as jnp
from jax import lax
from jax.experimental import pallas as pl
from jax.experimental.pallas import tpu as pltpu
from jax.experimental.pallas import tpu_sc as plsc

N, D, E = 10000, 128, 320000
NC, NS, L = 2, 16, 16
NW = NC * NS
C = 112
CHUNKS = 90
PER_W = CHUNKS * C
E_PAD = NW * PER_W
NCHG = NW * CHUNKS
N_PAD = 10112
ROWS_PER_TILE = N_PAD // NS
NBUF = 3
NI = 6


def _sc_body(h_hbm, packed_hbm, e_hbm, out_hbm,
             idx6, ev6, rows3, acc_sh, idx_sem, gat_sem, sct_sem):
    cid = lax.axis_index("c")
    sid = lax.axis_index("s")
    wid = sid * NC + cid

    def zero_body(i, _):
        rows3[0, i // (D // L), pl.ds((i % (D // L)) * L, L)] = (
            jnp.zeros((L,), jnp.float32))
        return 0

    lax.fori_loop(0, C * (D // L), zero_body, 0)
    row0 = sid * ROWS_PER_TILE
    for k in range(ROWS_PER_TILE // C):
        pltpu.sync_copy(rows3.at[0], acc_sh.at[pl.ds(row0 + k * C, C)])
    rem = ROWS_PER_TILE - (ROWS_PER_TILE // C) * C
    pltpu.sync_copy(rows3.at[0, pl.ds(0, rem)],
                    acc_sh.at[pl.ds(row0 + (ROWS_PER_TILE // C) * C, rem)])
    plsc.subcore_barrier()

    cbase = wid * CHUNKS

    def issue_idx(k, b):
        pltpu.async_copy(packed_hbm.at[cbase + k], idx6.at[b], idx_sem)
        pltpu.async_copy(e_hbm.at[pl.ds((cbase + k) * C, C)], ev6.at[b],
                         idx_sem)

    def wait_idx(b):
        pltpu.make_async_copy(packed_hbm.at[0], idx6.at[b], idx_sem).wait()
        pltpu.make_async_copy(e_hbm.at[pl.ds(0, C)], ev6.at[b],
                              idx_sem).wait()

    def issue_gat(ib, rb):
        pltpu.async_copy(h_hbm.at[idx6.at[ib, 0]], rows3.at[rb], gat_sem)

    def wait_gat(rb):
        pltpu.make_async_copy(h_hbm.at[pl.ds(0, C)], rows3.at[rb],
                              gat_sem).wait()

    def issue_sct(ib, rb):
        pltpu.async_copy(rows3.at[rb], acc_sh.at[idx6.at[ib, 1]], sct_sem,
                         add=True)

    def wait_sct():
        pltpu.make_async_copy(h_hbm.at[pl.ds(0, C)], rows3.at[0],
                              sct_sem).wait()

    def scale(ib, rb):
        def group_body(g, _):
            ev = ev6[ib, pl.ds(g * L, L)]
            for j in range(L):
                i = g * L + j
                eb = lax.gather(
                    ev, jnp.full((L, 1), j, jnp.int32),
                    lax.GatherDimensionNumbers(
                        offset_dims=(), collapsed_slice_dims=(0,),
                        start_index_map=(0,)),
                    slice_sizes=(1,),
                    mode=lax.GatherScatterMode.PROMISE_IN_BOUNDS)
                for f in range(D // L):
                    rows3[rb, i, pl.ds(f * L, L)] = (
                        rows3[rb, i, pl.ds(f * L, L)] * eb)
            return 0

        lax.fori_loop(0, C // L, group_body, 0)

    issue_idx(0, 0)
    issue_idx(1, 1)
    issue_idx(2, 2)
    wait_idx(0)
    issue_gat(0, 0)
    wait_idx(1)
    issue_gat(1, 1)

    def outer(g, _):
        for j in range(NI):
            k = g * NI + j
            rb = j % NBUF
            ib = j
            ib2 = (j + 2) % NI
            ib3 = (j + 3) % NI
            rb2 = (j + 2) % NBUF

            wait_gat(rb)
            scale(ib, rb)
            issue_sct(ib, rb)

            @pl.when(k <= CHUNKS - 4)
            def _():
                issue_idx(k + 3, ib3)

            @pl.when(k >= 1)
            def _():
                wait_sct()

            @pl.when(k <= CHUNKS - 3)
            def _():
                wait_idx(ib2)
                issue_gat(ib2, rb2)
        return 0

    lax.fori_loop(0, CHUNKS // NI, outer, 0)
    wait_sct()
    plsc.subcore_barrier()

    pltpu.sync_copy(acc_sh.at[pl.ds(row0, ROWS_PER_TILE)],
                    out_hbm.at[cid, pl.ds(row0, ROWS_PER_TILE)])


@jax.jit
def _sc_scatter(h, packed, e_p):
    mesh = plsc.VectorSubcoreMesh(core_axis_name="c", subcore_axis_name="s")
    return pl.kernel(
        _sc_body,
        out_type=jax.ShapeDtypeStruct((NC, N_PAD, D), jnp.float32),
        mesh=mesh,
        scratch_types=[
            pltpu.VMEM((NI, 2, C), jnp.int32),
            pltpu.VMEM((NI, C), jnp.float32),
            pltpu.VMEM((NBUF, C, D), jnp.float32),
            pltpu.VMEM_SHARED((N_PAD, D), jnp.float32),
            pltpu.SemaphoreType.DMA,
            pltpu.SemaphoreType.DMA,
            pltpu.SemaphoreType.DMA,
        ],
    )(h, packed, e_p)


def _tc_body(alpha_ref, parts_ref, h_ref, out_ref):
    s = jax.nn.sigmoid(alpha_ref[0, 0])
    out_ref[...] = s * (parts_ref[0, :N] + parts_ref[1, :N] - h_ref[...])


@jax.jit
def _tc_combine(alpha, parts, h):
    return pl.pallas_call(
        _tc_body,
        out_shape=jax.ShapeDtypeStruct((N, D), jnp.float32),
        in_specs=[
            pl.BlockSpec(memory_space=pltpu.SMEM),
            pl.BlockSpec(memory_space=pltpu.VMEM),
            pl.BlockSpec(memory_space=pltpu.VMEM),
        ],
        out_specs=pl.BlockSpec(memory_space=pltpu.VMEM),
    )(alpha, parts, h)


def kernel(t, x, edge_index, alpha):
    h = x[: N * D].reshape(N, D)
    e = x[N * D:]
    src = edge_index[0]
    dst = edge_index[1]
    pad = E_PAD - E
    src_p = jnp.concatenate([src, jnp.zeros((pad,), jnp.int32)])
    dst_p = jnp.concatenate([dst, jnp.zeros((pad,), jnp.int32)])
    e_p = jnp.concatenate([e, jnp.zeros((pad,), jnp.float32)])
    packed = (jnp.stack([src_p, dst_p], axis=0)
              .reshape(2, NCHG, C).transpose(1, 0, 2))
    parts = _sc_scatter(h, packed, e_p)
    h_new = _tc_combine(jnp.reshape(alpha, (1, 1)), parts, h)
    return jnp.concatenate([h_new.reshape(-1), jnp.zeros((E,), jnp.float32)])


# speedup vs baseline: 1.5991x; 1.5991x over previous
import jax
import jax.numpy as jnp
from jax import lax
from jax.experimental import pallas as pl
from jax.experimental.pallas import tpu as pltpu
from jax.experimental.pallas import tpu_sc as plsc

N, D, E = 10000, 128, 320000
NC, NS, L = 2, 16, 16
NW = NC * NS
C = 80
CHUNKS = E // (NW * C)
PER_W = CHUNKS * C
NCHG = NW * CHUNKS
N_PAD = 10240
ROWS_PER_TILE = N_PAD // NS
NBUF = 4
NI = 8
OUTER = -(-CHUNKS // NI)


def _sc_body(h_hbm, packed_hbm, e_hbm, out_hbm,
             idx8, ev8, rows4, acc_sh, idx_sem, gat_sem, sct_sem):
    cid = lax.axis_index("c")
    sid = lax.axis_index("s")
    wid = sid * NC + cid

    def zero_body(i, _):
        for f in range(D // L):
            rows4[0, i, pl.ds(f * L, L)] = jnp.zeros((L,), jnp.float32)
        return 0

    lax.fori_loop(0, C, zero_body, 0)
    row0 = sid * ROWS_PER_TILE
    for k in range(ROWS_PER_TILE // C):
        pltpu.sync_copy(rows4.at[0], acc_sh.at[pl.ds(row0 + k * C, C)])
    plsc.subcore_barrier()

    cbase = wid * CHUNKS

    def issue_idx(k, b):
        pltpu.async_copy(packed_hbm.at[cbase + k], idx8.at[b], idx_sem)
        pltpu.async_copy(e_hbm.at[pl.ds((cbase + k) * C, C)], ev8.at[b],
                         idx_sem)

    def wait_idx(b):
        pltpu.make_async_copy(packed_hbm.at[0], idx8.at[b], idx_sem).wait()
        pltpu.make_async_copy(e_hbm.at[pl.ds(0, C)], ev8.at[b],
                              idx_sem).wait()

    def issue_gat(ib, rb):
        pltpu.async_copy(h_hbm.at[idx8.at[ib, 0]], rows4.at[rb], gat_sem)

    def wait_gat(rb):
        pltpu.make_async_copy(h_hbm.at[pl.ds(0, C)], rows4.at[rb],
                              gat_sem).wait()

    def issue_sct(ib, rb):
        pltpu.async_copy(rows4.at[rb], acc_sh.at[idx8.at[ib, 1]], sct_sem,
                         add=True)

    def wait_sct():
        pltpu.make_async_copy(h_hbm.at[pl.ds(0, C)], rows4.at[0],
                              sct_sem).wait()

    def scale(ib, rb):
        def group_body(g, _):
            ev = ev8[ib, pl.ds(g * L, L)]
            for j in range(L):
                i = g * L + j
                eb = ev[j]
                for f in range(D // L):
                    rows4[rb, i, pl.ds(f * L, L)] = (
                        rows4[rb, i, pl.ds(f * L, L)] * eb)
            return 0

        lax.fori_loop(0, C // L, group_body, 0)

    for b in range(4):
        issue_idx(b, b)
    wait_idx(0)
    issue_gat(0, 0)
    wait_idx(1)
    issue_gat(1, 1)

    def outer(g, _):
        for j in range(NI):
            k = g * NI + j
            rb = j % NBUF
            ib = j
            ib2 = (j + 2) % NI
            ib4 = (j + 4) % NI
            rb2 = (j + 2) % NBUF

            @pl.when(k <= CHUNKS - 1)
            def _():
                wait_gat(rb)
                scale(ib, rb)
                issue_sct(ib, rb)

            @pl.when(k <= CHUNKS - 5)
            def _():
                issue_idx(k + 4, ib4)

            @pl.when((k >= 2) & (k <= CHUNKS + 1))
            def _():
                wait_sct()

            @pl.when(k <= CHUNKS - 3)
            def _():
                wait_idx(ib2)
                issue_gat(ib2, rb2)
        return 0

    lax.fori_loop(0, OUTER, outer, 0)
    plsc.subcore_barrier()

    pltpu.sync_copy(acc_sh.at[pl.ds(row0, ROWS_PER_TILE)],
                    out_hbm.at[cid, pl.ds(row0, ROWS_PER_TILE)])


@jax.jit
def _sc_scatter(h, packed, e_p):
    mesh = plsc.VectorSubcoreMesh(core_axis_name="c", subcore_axis_name="s")
    return pl.kernel(
        _sc_body,
        out_type=jax.ShapeDtypeStruct((NC, N_PAD, D), jnp.float32),
        mesh=mesh,
        scratch_types=[
            pltpu.VMEM((NI, 2, C), jnp.int32),
            pltpu.VMEM((NI, C), jnp.float32),
            pltpu.VMEM((NBUF, C, D), jnp.float32),
            pltpu.VMEM_SHARED((N_PAD, D), jnp.float32),
            pltpu.SemaphoreType.DMA,
            pltpu.SemaphoreType.DMA,
            pltpu.SemaphoreType.DMA,
        ],
    )(h, packed, e_p)


def _tc_body(alpha_ref, parts_ref, h_ref, out_ref):
    s = jax.nn.sigmoid(alpha_ref[0, 0])
    out_ref[...] = s * (parts_ref[0, :N] + parts_ref[1, :N] - h_ref[...])


@jax.jit
def _tc_combine(alpha, parts, h):
    return pl.pallas_call(
        _tc_body,
        out_shape=jax.ShapeDtypeStruct((N, D), jnp.float32),
        in_specs=[
            pl.BlockSpec(memory_space=pltpu.SMEM),
            pl.BlockSpec(memory_space=pltpu.VMEM),
            pl.BlockSpec(memory_space=pltpu.VMEM),
        ],
        out_specs=pl.BlockSpec(memory_space=pltpu.VMEM),
    )(alpha, parts, h)


def kernel(t, x, edge_index, alpha):
    h = x[: N * D].reshape(N, D)
    e = x[N * D:]
    src = edge_index[0]
    dst = edge_index[1]
    packed = (jnp.stack([src, dst], axis=0)
              .reshape(2, NCHG, C).transpose(1, 0, 2))
    parts = _sc_scatter(h, packed, e)
    h_new = _tc_combine(jnp.reshape(alpha, (1, 1)), parts, h)
    return jnp.concatenate([h_new.reshape(-1), jnp.zeros((E,), jnp.float32)])
